# baseline (speedup 1.0000x reference)
import functools

import jax
import jax.numpy as jnp
from jax import lax
from jax.experimental import pallas as pl
from jax.experimental.pallas import tpu as pltpu
from jax.experimental.pallas import tpu_sc as plsc

_B = 4
_N = 2048
_D = 64
_K = 32
_M = 512
_R2 = 0.2 * 0.2
_H1 = 128
_H2 = 256
_HG = 256
_E = _B * _M * _K
_NG = _B * _N
_MG = _B * _M
_INF = float("inf")

_NW = 32
_EPW = _E // _NW
_ECH = 512
_GPW = _MG // _NW


def _fps_body(p3_ref, idx_ref, ps3_ref, dists_ref):
    p3 = p3_ref[:, :, :]
    dists_ref[:, :] = jnp.full((_B, _N), _INF, jnp.float32)
    coln = lax.broadcasted_iota(jnp.int32, (_B, _N), 1)
    colm = lax.broadcasted_iota(jnp.int32, (_B, _M), 1)
    rowoff = lax.broadcasted_iota(jnp.int32, (_B, 1), 0) * _N

    def step(i, carry):
        last, ai, aps = carry
        oh = (coln == last).astype(jnp.float32)
        c3 = jnp.sum(p3 * oh[None, :, :], axis=2, keepdims=True)
        d = ((p3[0] - c3[0]) ** 2 + (p3[1] - c3[1]) ** 2
             + (p3[2] - c3[2]) ** 2)
        nd = jnp.minimum(dists_ref[:, :], d)
        dists_ref[:, :] = nd
        mx = jnp.max(nd, axis=1, keepdims=True)
        nxt = jnp.min(jnp.where(nd == mx, coln, _N), axis=1, keepdims=True)
        at = colm == i
        return (nxt, jnp.where(at, last + rowoff, ai),
                jnp.where(at[None, :, :], c3, aps))

    _, ai, aps = lax.fori_loop(
        0, _M, step,
        (jnp.zeros((_B, 1), jnp.int32), jnp.zeros((_B, _M), jnp.int32),
         jnp.zeros((3, _B, _M), jnp.float32)))
    idx_ref[:, :] = ai
    ps3_ref[:, :, :] = aps


_fps = pl.pallas_call(
    _fps_body,
    out_shape=[
        jax.ShapeDtypeStruct((_B, _M), jnp.int32),
        jax.ShapeDtypeStruct((3, _B, _M), jnp.float32),
    ],
    scratch_shapes=[pltpu.VMEM((_B, _N), jnp.float32)],
)


def _sel_body(px_ref, py_ref, pz_ref, psx_ref, psy_ref, psz_ref,
              nidx_ref, valid_ref, d2_ref):
    b = pl.program_id(0)
    px = px_ref[0, 0, :][None, :]
    py = py_ref[0, 0, :][None, :]
    pz = pz_ref[0, 0, :][None, :]
    cx = psx_ref[0]
    cy = psy_ref[0]
    cz = psz_ref[0]
    d2 = (cx - px) ** 2 + (cy - py) ** 2 + (cz - pz) ** 2
    d2_ref[:, :] = jnp.where(d2 <= _R2, d2, _INF)
    coln = lax.broadcasted_iota(jnp.int32, (_M, _N), 1)
    colk = lax.broadcasted_iota(jnp.int32, (_M, _K), 1)

    def step(k, carry):
        an, av = carry
        cur = d2_ref[:, :]
        mn = jnp.min(cur, axis=1, keepdims=True)
        sel = jnp.min(jnp.where(cur == mn, coln, _N), axis=1, keepdims=True)
        d2_ref[:, :] = jnp.where(coln == sel, _INF, cur)
        at = colk == k
        return (jnp.where(at, sel + b * _N, an),
                jnp.where(at, (mn <= _R2).astype(jnp.int32), av))

    zi = jnp.zeros((_M, _K), jnp.int32)
    an, av = lax.fori_loop(0, _K, step, (zi, zi))
    nidx_ref[0, :, :] = an
    valid_ref[0, :, :] = av


_sel = pl.pallas_call(
    _sel_body,
    grid=(_B,),
    in_specs=[
        pl.BlockSpec((1, 1, _N), lambda b: (b, 0, 0)),
        pl.BlockSpec((1, 1, _N), lambda b: (b, 0, 0)),
        pl.BlockSpec((1, 1, _N), lambda b: (b, 0, 0)),
        pl.BlockSpec((1, _M, 1), lambda b: (b, 0, 0)),
        pl.BlockSpec((1, _M, 1), lambda b: (b, 0, 0)),
        pl.BlockSpec((1, _M, 1), lambda b: (b, 0, 0)),
    ],
    out_specs=[
        pl.BlockSpec((1, _M, _K), lambda b: (b, 0, 0)),
        pl.BlockSpec((1, _M, _K), lambda b: (b, 0, 0)),
    ],
    out_shape=[
        jax.ShapeDtypeStruct((_B, _M, _K), jnp.int32),
        jax.ShapeDtypeStruct((_B, _M, _K), jnp.int32),
    ],
    scratch_shapes=[pltpu.VMEM((_M, _N), jnp.float32)],
)


def _sc_gather_body(x_hbm, posp_hbm, psp_hbm, eidx_hbm, cidx_hbm, ig_hbm,
                    xg_hbm, relp_hbm, po_hbm,
                    eidx_v, cidx_v, xbuf, pjbuf, psbuf, relbuf,
                    igv, pobuf, sem):
    wid = lax.axis_index("s") * 2 + lax.axis_index("c")
    base = wid * _EPW
    pltpu.sync_copy(eidx_hbm.at[pl.ds(base, _EPW)], eidx_v)
    pltpu.sync_copy(cidx_hbm.at[pl.ds(base, _EPW)], cidx_v)

    for ch in range(_EPW // _ECH):
        esl = eidx_v.at[pl.ds(ch * _ECH, _ECH)]
        csl = cidx_v.at[pl.ds(ch * _ECH, _ECH)]
        pltpu.async_copy(x_hbm.at[esl], xbuf, sem).wait()
        pltpu.sync_copy(xbuf, xg_hbm.at[pl.ds(base + ch * _ECH, _ECH)])
        pltpu.async_copy(posp_hbm.at[esl], pjbuf, sem).wait()
        pltpu.async_copy(psp_hbm.at[csl], psbuf, sem).wait()

        def sub(i, _):
            relbuf[i, :] = pjbuf[i, :] - psbuf[i, :]
            return 0

        lax.fori_loop(0, _ECH, sub, 0)
        pltpu.sync_copy(relbuf, relp_hbm.at[pl.ds(base + ch * _ECH, _ECH)])

    gbase = wid * _GPW
    pltpu.sync_copy(ig_hbm.at[pl.ds(gbase, _GPW)], igv)
    pltpu.async_copy(posp_hbm.at[igv], pobuf, sem).wait()
    pltpu.sync_copy(pobuf, po_hbm.at[pl.ds(gbase, _GPW)])


@functools.lru_cache(maxsize=1)
def _make_sc_gather():
    return functools.partial(
        pl.kernel,
        mesh=plsc.VectorSubcoreMesh(core_axis_name="c", subcore_axis_name="s"),
        compiler_params=pltpu.CompilerParams(use_tc_tiling_on_sc=False),
        out_type=[
            jax.ShapeDtypeStruct((_E, _D), jnp.float32),
            jax.ShapeDtypeStruct((_E, 16), jnp.float32),
            jax.ShapeDtypeStruct((_MG, 16), jnp.float32),
        ],
        scratch_types=[
            pltpu.VMEM((_EPW,), jnp.int32),
            pltpu.VMEM((_EPW,), jnp.int32),
            pltpu.VMEM((_ECH, _D), jnp.float32),
            pltpu.VMEM((_ECH, 16), jnp.float32),
            pltpu.VMEM((_ECH, 16), jnp.float32),
            pltpu.VMEM((_ECH, 16), jnp.float32),
            pltpu.VMEM((_GPW,), jnp.int32),
            pltpu.VMEM((_GPW, 16), jnp.float32),
            pltpu.SemaphoreType.DMA,
        ],
    )(_sc_gather_body)


_MB = 256
_EB = _MB * _K


def _mlp_body(xg_ref, rel_ref, val_ref, w1x_ref, w1p_ref, b1_ref,
              w2_ref, b2_ref, wg_ref, bg_ref, out_ref):
    dot = functools.partial(lax.dot_general,
                            dimension_numbers=(((1,), (0,)), ((), ())),
                            preferred_element_type=jnp.float32,
                            precision=lax.Precision.DEFAULT)
    t = dot(xg_ref[:, :], w1x_ref[:, :]) + dot(rel_ref[:, :], w1p_ref[:, :])
    h1 = jnp.maximum(t + b1_ref[0, :][None, :], 0.0)
    h2 = jnp.maximum(dot(h1, w2_ref[:, :]) + b2_ref[0, :][None, :], 0.0)
    h2 = jnp.where(val_ref[:, :] > 0.0, h2, -1e10)
    agg = jnp.max(h2.reshape(_MB, _K, _H2), axis=1)
    out_ref[:, :] = jnp.maximum(dot(agg, wg_ref[:, :]) + bg_ref[0, :][None, :],
                                0.0)


_mlp = pl.pallas_call(
    _mlp_body,
    grid=(_E // _EB,),
    in_specs=[
        pl.BlockSpec((_EB, _D), lambda i: (i, 0)),
        pl.BlockSpec((_EB, 16), lambda i: (i, 0)),
        pl.BlockSpec((_EB, 1), lambda i: (i, 0)),
        pl.BlockSpec((_D, _H1), lambda i: (0, 0)),
        pl.BlockSpec((16, _H1), lambda i: (0, 0)),
        pl.BlockSpec((1, _H1), lambda i: (0, 0)),
        pl.BlockSpec((_H1, _H2), lambda i: (0, 0)),
        pl.BlockSpec((1, _H2), lambda i: (0, 0)),
        pl.BlockSpec((_H2, _HG), lambda i: (0, 0)),
        pl.BlockSpec((1, _HG), lambda i: (0, 0)),
    ],
    out_specs=pl.BlockSpec((_MB, _HG), lambda i: (i, 0)),
    out_shape=jax.ShapeDtypeStruct((_MG, _HG), jnp.float32),
)


def kernel(x, pos, batch, point2curveidx, W1, b1, W2, b2, Wg, bg):
    p3 = pos.T.reshape(3, _B, _N)

    idxg, ps3 = _fps(p3)

    nidx, valid = _sel(p3[0].reshape(_B, 1, _N),
                       p3[1].reshape(_B, 1, _N),
                       p3[2].reshape(_B, 1, _N),
                       ps3[0].reshape(_B, _M, 1),
                       ps3[1].reshape(_B, _M, 1),
                       ps3[2].reshape(_B, _M, 1))

    batf = batch.astype(jnp.float32).reshape(_NG, 1)
    curf = point2curveidx.astype(jnp.float32).reshape(_NG, 1)
    posp = jnp.concatenate(
        [pos, batf, curf, jnp.zeros((_NG, 11), jnp.float32)], axis=1)
    psp = jnp.pad(ps3.reshape(3, _MG).T, ((0, 0), (0, 13)))
    eidx = nidx.reshape(_E)
    cidx = jnp.repeat(jnp.arange(_MG, dtype=jnp.int32), _K)
    igf = idxg.reshape(_MG)

    xg, relp, pop = _make_sc_gather()(x, posp, psp, eidx, cidx, igf)

    w1x = W1[:_D]
    w1p = jnp.zeros((16, _H1), W1.dtype).at[:3].set(W1[_D:])
    validf = valid.reshape(_E, 1).astype(jnp.float32)

    x_out = _mlp(xg, relp, validf, w1x, w1p, b1.reshape(1, _H1),
                 W2, b2.reshape(1, _H2), Wg, bg.reshape(1, _HG))

    pos_out = pop[:, :3]
    return (x_out, pos_out, pop[:, 3].astype(batch.dtype),
            pop[:, 4].astype(point2curveidx.dtype), igf)


# speedup vs baseline: 1.0071x; 1.0071x over previous
import functools

import jax
import jax.numpy as jnp
from jax import lax
from jax.experimental import pallas as pl
from jax.experimental.pallas import tpu as pltpu
from jax.experimental.pallas import tpu_sc as plsc

_B = 4
_N = 2048
_D = 64
_K = 32
_M = 512
_R2 = 0.2 * 0.2
_H1 = 128
_H2 = 256
_HG = 256
_E = _B * _M * _K
_NG = _B * _N
_MG = _B * _M
_INF = float("inf")

_NW = 32
_EPW = _E // _NW
_ECH = 512
_GPW = _MG // _NW


def _fps_body(p3_ref, idx_ref, ps3_ref, dists_ref):
    p3 = p3_ref[:, :, :]
    dists_ref[:, :] = jnp.full((_B, _N), _INF, jnp.float32)
    coln = lax.broadcasted_iota(jnp.int32, (_B, _N), 1)
    colm = lax.broadcasted_iota(jnp.int32, (_B, _M), 1)
    rowoff = lax.broadcasted_iota(jnp.int32, (_B, 1), 0) * _N

    def step(i, carry):
        last, ai, aps = carry
        oh = (coln == last).astype(jnp.float32)
        c3 = jnp.sum(p3 * oh[None, :, :], axis=2, keepdims=True)
        d = ((p3[0] - c3[0]) ** 2 + (p3[1] - c3[1]) ** 2
             + (p3[2] - c3[2]) ** 2)
        nd = jnp.minimum(dists_ref[:, :], d)
        dists_ref[:, :] = nd
        mx = jnp.max(nd, axis=1, keepdims=True)
        nxt = jnp.min(jnp.where(nd == mx, coln, _N), axis=1, keepdims=True)
        at = colm == i
        return (nxt, jnp.where(at, last + rowoff, ai),
                jnp.where(at[None, :, :], c3, aps))

    _, ai, aps = lax.fori_loop(
        0, _M, step,
        (jnp.zeros((_B, 1), jnp.int32), jnp.zeros((_B, _M), jnp.int32),
         jnp.zeros((3, _B, _M), jnp.float32)))
    idx_ref[:, :] = ai
    ps3_ref[:, :, :] = aps


_fps = pl.pallas_call(
    _fps_body,
    out_shape=[
        jax.ShapeDtypeStruct((_B, _M), jnp.int32),
        jax.ShapeDtypeStruct((3, _B, _M), jnp.float32),
    ],
    scratch_shapes=[pltpu.VMEM((_B, _N), jnp.float32)],
)


def _sel_body(px_ref, py_ref, pz_ref, psx_ref, psy_ref, psz_ref,
              nidx_ref, valid_ref, d2_ref):
    b = pl.program_id(0)
    px = px_ref[0, 0, :][None, :]
    py = py_ref[0, 0, :][None, :]
    pz = pz_ref[0, 0, :][None, :]
    cx = psx_ref[0]
    cy = psy_ref[0]
    cz = psz_ref[0]
    d2 = (cx - px) ** 2 + (cy - py) ** 2 + (cz - pz) ** 2
    d2_ref[:, :] = jnp.where(d2 <= _R2, d2, _INF)
    coln = lax.broadcasted_iota(jnp.int32, (_M, _N), 1)
    colk = lax.broadcasted_iota(jnp.int32, (_M, _K), 1)

    def step(k, carry):
        an, av = carry
        cur = d2_ref[:, :]
        mn = jnp.min(cur, axis=1, keepdims=True)
        sel = jnp.min(jnp.where(cur == mn, coln, _N), axis=1, keepdims=True)
        d2_ref[:, :] = jnp.where(coln == sel, _INF, cur)
        at = colk == k
        return (jnp.where(at, sel + b * _N, an),
                jnp.where(at, (mn <= _R2).astype(jnp.int32), av))

    zi = jnp.zeros((_M, _K), jnp.int32)
    an, av = lax.fori_loop(0, _K, step, (zi, zi))
    nidx_ref[0, :, :] = an
    valid_ref[0, :, :] = av


_sel = pl.pallas_call(
    _sel_body,
    grid=(_B,),
    in_specs=[
        pl.BlockSpec((1, 1, _N), lambda b: (b, 0, 0)),
        pl.BlockSpec((1, 1, _N), lambda b: (b, 0, 0)),
        pl.BlockSpec((1, 1, _N), lambda b: (b, 0, 0)),
        pl.BlockSpec((1, _M, 1), lambda b: (b, 0, 0)),
        pl.BlockSpec((1, _M, 1), lambda b: (b, 0, 0)),
        pl.BlockSpec((1, _M, 1), lambda b: (b, 0, 0)),
    ],
    out_specs=[
        pl.BlockSpec((1, _M, _K), lambda b: (b, 0, 0)),
        pl.BlockSpec((1, _M, _K), lambda b: (b, 0, 0)),
    ],
    out_shape=[
        jax.ShapeDtypeStruct((_B, _M, _K), jnp.int32),
        jax.ShapeDtypeStruct((_B, _M, _K), jnp.int32),
    ],
    scratch_shapes=[pltpu.VMEM((_M, _N), jnp.float32)],
)


def _sc_gather_body(x_hbm, posp_hbm, psp_hbm, eidx_hbm, cidx_hbm, ig_hbm,
                    xg_hbm, relp_hbm, po_hbm,
                    eidx_v, cidx_v, xbuf0, xbuf1, pjbuf, psbuf, relbuf,
                    igv, pobuf, semx0, semx1, semp, semo):
    wid = lax.axis_index("s") * 2 + lax.axis_index("c")
    base = wid * _EPW
    pltpu.sync_copy(eidx_hbm.at[pl.ds(base, _EPW)], eidx_v)
    pltpu.sync_copy(cidx_hbm.at[pl.ds(base, _EPW)], cidx_v)

    gbase = wid * _GPW
    pltpu.sync_copy(ig_hbm.at[pl.ds(gbase, _GPW)], igv)
    ocp = pltpu.async_copy(posp_hbm.at[igv], pobuf, semo)

    nch = _EPW // _ECH
    xbufs = (xbuf0, xbuf1)
    xsems = (semx0, semx1)
    cps = [None, None]
    cps[0] = pltpu.async_copy(x_hbm.at[eidx_v.at[pl.ds(0, _ECH)]],
                              xbufs[0], xsems[0])
    for ch in range(nch):
        esl = eidx_v.at[pl.ds(ch * _ECH, _ECH)]
        csl = cidx_v.at[pl.ds(ch * _ECH, _ECH)]
        if ch + 1 < nch:
            nsl = eidx_v.at[pl.ds((ch + 1) * _ECH, _ECH)]
            cps[(ch + 1) % 2] = pltpu.async_copy(
                x_hbm.at[nsl], xbufs[(ch + 1) % 2], xsems[(ch + 1) % 2])
        cpj = pltpu.async_copy(posp_hbm.at[esl], pjbuf, semp)
        cps_ = pltpu.async_copy(psp_hbm.at[csl], psbuf, semp)
        cpj.wait()
        cps_.wait()

        def sub(i, _):
            relbuf[i, :] = pjbuf[i, :] - psbuf[i, :]
            return 0

        lax.fori_loop(0, _ECH, sub, 0)
        pltpu.sync_copy(relbuf, relp_hbm.at[pl.ds(base + ch * _ECH, _ECH)])
        cps[ch % 2].wait()
        pltpu.sync_copy(xbufs[ch % 2], xg_hbm.at[pl.ds(base + ch * _ECH, _ECH)])

    ocp.wait()
    pltpu.sync_copy(pobuf, po_hbm.at[pl.ds(gbase, _GPW)])


@functools.lru_cache(maxsize=1)
def _make_sc_gather():
    return functools.partial(
        pl.kernel,
        mesh=plsc.VectorSubcoreMesh(core_axis_name="c", subcore_axis_name="s"),
        compiler_params=pltpu.CompilerParams(use_tc_tiling_on_sc=False),
        out_type=[
            jax.ShapeDtypeStruct((_E, _D), jnp.float32),
            jax.ShapeDtypeStruct((_E, 16), jnp.float32),
            jax.ShapeDtypeStruct((_MG, 16), jnp.float32),
        ],
        scratch_types=[
            pltpu.VMEM((_EPW,), jnp.int32),
            pltpu.VMEM((_EPW,), jnp.int32),
            pltpu.VMEM((_ECH, _D), jnp.float32),
            pltpu.VMEM((_ECH, _D), jnp.float32),
            pltpu.VMEM((_ECH, 16), jnp.float32),
            pltpu.VMEM((_ECH, 16), jnp.float32),
            pltpu.VMEM((_ECH, 16), jnp.float32),
            pltpu.VMEM((_GPW,), jnp.int32),
            pltpu.VMEM((_GPW, 16), jnp.float32),
            pltpu.SemaphoreType.DMA,
            pltpu.SemaphoreType.DMA,
            pltpu.SemaphoreType.DMA,
            pltpu.SemaphoreType.DMA,
        ],
    )(_sc_gather_body)


_MB = 256
_EB = _MB * _K


def _mlp_body(xg_ref, rel_ref, val_ref, w1x_ref, w1p_ref, b1_ref,
              w2_ref, b2_ref, wg_ref, bg_ref, out_ref):
    dot = functools.partial(lax.dot_general,
                            dimension_numbers=(((1,), (0,)), ((), ())),
                            preferred_element_type=jnp.float32,
                            precision=lax.Precision.DEFAULT)
    t = dot(xg_ref[:, :], w1x_ref[:, :]) + dot(rel_ref[:, :], w1p_ref[:, :])
    h1 = jnp.maximum(t + b1_ref[0, :][None, :], 0.0)
    h2 = jnp.maximum(dot(h1, w2_ref[:, :]) + b2_ref[0, :][None, :], 0.0)
    h2 = jnp.where(val_ref[:, :] > 0.0, h2, -1e10)
    agg = jnp.max(h2.reshape(_MB, _K, _H2), axis=1)
    out_ref[:, :] = jnp.maximum(dot(agg, wg_ref[:, :]) + bg_ref[0, :][None, :],
                                0.0)


_mlp = pl.pallas_call(
    _mlp_body,
    grid=(_E // _EB,),
    in_specs=[
        pl.BlockSpec((_EB, _D), lambda i: (i, 0)),
        pl.BlockSpec((_EB, 16), lambda i: (i, 0)),
        pl.BlockSpec((_EB, 1), lambda i: (i, 0)),
        pl.BlockSpec((_D, _H1), lambda i: (0, 0)),
        pl.BlockSpec((16, _H1), lambda i: (0, 0)),
        pl.BlockSpec((1, _H1), lambda i: (0, 0)),
        pl.BlockSpec((_H1, _H2), lambda i: (0, 0)),
        pl.BlockSpec((1, _H2), lambda i: (0, 0)),
        pl.BlockSpec((_H2, _HG), lambda i: (0, 0)),
        pl.BlockSpec((1, _HG), lambda i: (0, 0)),
    ],
    out_specs=pl.BlockSpec((_MB, _HG), lambda i: (i, 0)),
    out_shape=jax.ShapeDtypeStruct((_MG, _HG), jnp.float32),
)


def kernel(x, pos, batch, point2curveidx, W1, b1, W2, b2, Wg, bg):
    p3 = pos.T.reshape(3, _B, _N)

    idxg, ps3 = _fps(p3)

    nidx, valid = _sel(p3[0].reshape(_B, 1, _N),
                       p3[1].reshape(_B, 1, _N),
                       p3[2].reshape(_B, 1, _N),
                       ps3[0].reshape(_B, _M, 1),
                       ps3[1].reshape(_B, _M, 1),
                       ps3[2].reshape(_B, _M, 1))

    batf = batch.astype(jnp.float32).reshape(_NG, 1)
    curf = point2curveidx.astype(jnp.float32).reshape(_NG, 1)
    posp = jnp.concatenate(
        [pos, batf, curf, jnp.zeros((_NG, 11), jnp.float32)], axis=1)
    psp = jnp.pad(ps3.reshape(3, _MG).T, ((0, 0), (0, 13)))
    eidx = nidx.reshape(_E)
    cidx = jnp.repeat(jnp.arange(_MG, dtype=jnp.int32), _K)
    igf = idxg.reshape(_MG)

    xg, relp, pop = _make_sc_gather()(x, posp, psp, eidx, cidx, igf)

    w1x = W1[:_D]
    w1p = jnp.zeros((16, _H1), W1.dtype).at[:3].set(W1[_D:])
    validf = valid.reshape(_E, 1).astype(jnp.float32)

    x_out = _mlp(xg, relp, validf, w1x, w1p, b1.reshape(1, _H1),
                 W2, b2.reshape(1, _H2), Wg, bg.reshape(1, _HG))

    pos_out = pop[:, :3]
    return (x_out, pos_out, pop[:, 3].astype(batch.dtype),
            pop[:, 4].astype(point2curveidx.dtype), igf)
